# baseline (speedup 1.0000x reference)
import functools

import jax
import jax.numpy as jnp
from jax import lax
from jax.experimental import pallas as pl
from jax.experimental.pallas import tpu as pltpu
from jax.experimental.pallas import tpu_sc as plsc

_POSE_NUM = 100000
_EMBED_DIM = 6
_BATCH = 16384

_info = plsc.get_sparse_core_info()
_NC, _NS = _info.num_cores, _info.num_subcores
_NW = _NC * _NS
_B_PER_W = _BATCH // _NW
_E_PER_W = _B_PER_W * _EMBED_DIM
_B_CHUNK = _BATCH // (_NW // 2)


def _make_gather():
    mesh = plsc.VectorSubcoreMesh(core_axis_name="c", subcore_axis_name="s")

    @functools.partial(
        pl.kernel,
        mesh=mesh,
        out_type=jax.ShapeDtypeStruct((_BATCH * _EMBED_DIM,), jnp.float32),
        scratch_types=[
            pltpu.VMEM((_B_CHUNK,), jnp.int32),
            pltpu.VMEM((3 * _B_CHUNK,), jnp.float32),
            pltpu.SemaphoreType.DMA,
            pltpu.SemaphoreType.DMA,
        ],
        compiler_params=pltpu.CompilerParams(
            use_tc_tiling_on_sc=False, needs_layout_passes=False
        ),
    )
    def gather_kernel(idx_hbm, tab_cm_hbm, out_hbm, idx_v, vals_v, gsem, osem):
        wid = lax.axis_index("s") * _NC + lax.axis_index("c")
        chunk = wid % (_NW // 2)
        j0 = wid // (_NW // 2)
        base = chunk * _B_CHUNK
        pltpu.sync_copy(idx_hbm.at[pl.ds(base, _B_CHUNK)], idx_v)

        gathers = []
        for k in range(3):
            j = j0 + 2 * k
            col = tab_cm_hbm.at[pl.ds(j * _POSE_NUM, _POSE_NUM)]
            gathers.append(
                pltpu.async_copy(
                    col.at[idx_v],
                    vals_v.at[pl.ds(k * _B_CHUNK, _B_CHUNK)],
                    gsem,
                )
            )
        stores = []
        for k in range(3):
            j = j0 + 2 * k
            gathers[k].wait()
            stores.append(
                pltpu.async_copy(
                    vals_v.at[pl.ds(k * _B_CHUNK, _B_CHUNK)],
                    out_hbm.at[pl.ds(j * _BATCH + base, _B_CHUNK)],
                    osem,
                )
            )
        for s in stores:
            s.wait()

    return gather_kernel


_gather = _make_gather()


def kernel(indices, table):
    flat_cm = table.T.reshape(-1)
    out_cm = _gather(indices.astype(jnp.int32), flat_cm)
    return out_cm.reshape(_EMBED_DIM, _BATCH).T
